# baseline (speedup 1.0000x reference)
import functools

import jax
import jax.numpy as jnp
from jax import lax
from jax.experimental import pallas as pl
from jax.experimental.pallas import tpu as pltpu
from jax.experimental.pallas import tpu_sc as plsc

N = 10000
NP = 10240
E = 320000
G = 64
H = 16
NC = 2
NS = 16
NW = NC * NS
CH = 128
NCHUNK = 79
EPW = NCHUNK * CH
PAD_NODE = NP - 1
RPT = NP // NS

_mesh = plsc.VectorSubcoreMesh(core_axis_name="c", subcore_axis_name="s")
_sc_params = pltpu.CompilerParams(use_tc_tiling_on_sc=False)


@functools.partial(
    pl.kernel,
    out_type=jax.ShapeDtypeStruct((NC, NP), jnp.float32),
    mesh=_mesh,
    compiler_params=_sc_params,
    scratch_types=[
        pltpu.VMEM((NCHUNK, CH), jnp.int32),
        pltpu.VMEM((CH,), jnp.float32),
        pltpu.VMEM((RPT,), jnp.float32),
        pltpu.VMEM_SHARED((NP,), jnp.float32),
    ] + [pltpu.SemaphoreType.DMA] * 8,
)
def _deg_kernel(eidx_hbm, out_hbm, dst_v, ones_v, stage_v, acc_sh, *ssem):
    c = lax.axis_index("c")
    s = lax.axis_index("s")

    def fill(i, _):
        ones_v[pl.ds(i * 16, 16)] = jnp.ones((16,), jnp.float32)
        return 0

    lax.fori_loop(0, CH // 16, fill, 0)

    def zero(i, _):
        stage_v[pl.ds(i * 16, 16)] = jnp.zeros((16,), jnp.float32)
        return 0

    lax.fori_loop(0, RPT // 16, zero, 0)
    pltpu.sync_copy(stage_v, acc_sh.at[pl.ds(s * RPT, RPT)])
    plsc.subcore_barrier()

    wid = s * NC + c
    pltpu.sync_copy(eidx_hbm.at[1, wid], dst_v)

    def body(g, _):
        for b in range(8):
            j = g * 8 + b

            @pl.when(j < NCHUNK)
            def _():
                @pl.when(j >= 8)
                def _():
                    pltpu.make_async_copy(
                        ones_v, acc_sh.at[dst_v.at[0]], ssem[b]).wait()

                pltpu.async_copy(
                    ones_v, acc_sh.at[dst_v.at[j]], ssem[b], add=True)

        return 0

    lax.fori_loop(0, (NCHUNK + 7) // 8, body, 0)
    for b in range(8):
        pltpu.make_async_copy(ones_v, acc_sh.at[dst_v.at[0]], ssem[b]).wait()
    plsc.subcore_barrier()
    pltpu.sync_copy(acc_sh.at[pl.ds(s * RPT, RPT)], stage_v)
    pltpu.sync_copy(stage_v, out_hbm.at[c, pl.ds(s * RPT, RPT)])


def _edge_loop(y_ref, src_v, dst_v, rows, acc_sh, gsem, ssem):
    def wait_g(b):
        pltpu.make_async_copy(y_ref.at[src_v.at[0]], rows.at[b],
                              gsem[b]).wait()

    def wait_s(b):
        pltpu.make_async_copy(rows.at[b], acc_sh.at[dst_v.at[0]],
                              ssem[b]).wait()

    for b in range(6):
        pltpu.async_copy(y_ref.at[src_v.at[b]], rows.at[b], gsem[b])

    def body(g, _):
        for b in range(12):
            j = g * 12 + b
            bf = (b + 6) % 12

            @pl.when(j + 6 < NCHUNK)
            def _():
                @pl.when(j >= 6)
                def _():
                    wait_s(bf)

                pltpu.async_copy(
                    y_ref.at[src_v.at[j + 6]], rows.at[bf], gsem[bf])

            @pl.when(j < NCHUNK)
            def _():
                wait_g(b)
                pltpu.async_copy(
                    rows.at[b], acc_sh.at[dst_v.at[j]], ssem[b], add=True)

        return 0

    lax.fori_loop(0, (NCHUNK + 11) // 12, body, 0)
    for b in range(12):
        wait_s(b)


@functools.partial(
    pl.kernel,
    out_type=jax.ShapeDtypeStruct((NC, NP, H), jnp.float32),
    mesh=_mesh,
    compiler_params=_sc_params,
    scratch_types=[
        pltpu.VMEM((NCHUNK, CH), jnp.int32),
        pltpu.VMEM((NCHUNK, CH), jnp.int32),
        pltpu.VMEM((12, CH, H), jnp.float32),
        pltpu.VMEM((RPT, H), jnp.float32),
        pltpu.VMEM_SHARED((NP, H), jnp.float32),
    ] + [pltpu.SemaphoreType.DMA] * 26,
)
def _edge_kernel(y_hbm, eidx_hbm, out_hbm,
                 src_v, dst_v, rows, stage, acc_sh, *sems):
    gsem = sems[:12]
    ssem = sems[12:24]
    isem = sems[24:]
    c = lax.axis_index("c")
    s = lax.axis_index("s")

    wid = s * NC + c
    pltpu.async_copy(eidx_hbm.at[0, wid], src_v, isem[0])
    pltpu.async_copy(eidx_hbm.at[1, wid], dst_v, isem[1])

    def zero(i, _):
        stage[i, :] = jnp.zeros((H,), jnp.float32)
        return 0

    lax.fori_loop(0, RPT, zero, 0)
    pltpu.sync_copy(stage, acc_sh.at[pl.ds(s * RPT, RPT)])
    pltpu.make_async_copy(eidx_hbm.at[0, wid], src_v, isem[0]).wait()
    pltpu.make_async_copy(eidx_hbm.at[1, wid], dst_v, isem[1]).wait()
    plsc.subcore_barrier()

    _edge_loop(y_hbm, src_v, dst_v, rows, acc_sh, gsem, ssem)

    plsc.subcore_barrier()
    pltpu.sync_copy(acc_sh.at[pl.ds(s * RPT, RPT)], stage)
    pltpu.sync_copy(stage, out_hbm.at[c, pl.ds(s * RPT, RPT)])


@functools.partial(
    pl.kernel,
    out_type=(jax.ShapeDtypeStruct((NC, NP, H), jnp.float32),
              jax.ShapeDtypeStruct((NP, H), jnp.float32)),
    mesh=_mesh,
    compiler_params=_sc_params,
    scratch_types=[
        pltpu.VMEM((NCHUNK, CH), jnp.int32),
        pltpu.VMEM((NCHUNK, CH), jnp.int32),
        pltpu.VMEM((12, CH, H), jnp.float32),
        pltpu.VMEM((RPT, H), jnp.float32),
        pltpu.VMEM((RPT, H), jnp.float32),
        pltpu.VMEM((RPT, H), jnp.float32),
        pltpu.VMEM((RPT, H), jnp.float32),
        pltpu.VMEM((RPT, H), jnp.float32),
        pltpu.VMEM((RPT,), jnp.float32),
        pltpu.VMEM((16,), jnp.float32),
        pltpu.VMEM((H, H), jnp.float32),
        pltpu.VMEM_SHARED((NP, H), jnp.float32),
        pltpu.VMEM_SHARED((NP, H), jnp.float32),
    ] + [pltpu.SemaphoreType.DMA] * 27,
)
def _emid_kernel(yp_hbm, acc_hbm, dis_hbm, b_hbm, w_hbm, eidx_hbm,
                 out_hbm, ynext_hbm,
                 src_v, dst_v, rows, stage, yp_v, a0_v, a1_v, yn_v,
                 dis_v, b_v, w_v, y_sh, acc_sh, *sems):
    gsem = sems[:12]
    ssem = sems[12:24]
    isem = sems[24:]
    c = lax.axis_index("c")
    s = lax.axis_index("s")
    wid = s * NC + c
    row0 = s * RPT

    pltpu.async_copy(eidx_hbm.at[0, wid], src_v, isem[0])
    pltpu.async_copy(eidx_hbm.at[1, wid], dst_v, isem[1])
    pltpu.sync_copy(yp_hbm.at[pl.ds(row0, RPT)], yp_v)
    pltpu.sync_copy(acc_hbm.at[0, pl.ds(row0, RPT)], a0_v)
    pltpu.sync_copy(acc_hbm.at[1, pl.ds(row0, RPT)], a1_v)
    pltpu.sync_copy(dis_hbm.at[pl.ds(row0, RPT)], dis_v)
    pltpu.sync_copy(b_hbm, b_v)
    pltpu.sync_copy(w_hbm, w_v)
    wrows = [w_v[k, :] for k in range(H)]
    bvec = b_v[...]

    def dense(gi, _):
        i0 = gi * 16
        dvec = dis_v[pl.ds(i0, 16)]
        for r in range(16):
            i = i0 + r
            d = dvec[r]
            a = a0_v[i, :] + a1_v[i, :] + yp_v[i, :]
            h = jnp.maximum(a * d + bvec, 0.0)
            o = h[0] * wrows[0]
            for k in range(1, H):
                o = o + h[k] * wrows[k]
            yn_v[i, :] = o * d
        return 0

    lax.fori_loop(0, RPT // 16, dense, 0)
    pltpu.sync_copy(yn_v, y_sh.at[pl.ds(row0, RPT)])

    @pl.when(c == 0)
    def _():
        pltpu.async_copy(yn_v, ynext_hbm.at[pl.ds(row0, RPT)], isem[2])

    def zero(i, _):
        stage[i, :] = jnp.zeros((H,), jnp.float32)
        return 0

    lax.fori_loop(0, RPT, zero, 0)
    pltpu.sync_copy(stage, acc_sh.at[pl.ds(row0, RPT)])
    pltpu.make_async_copy(eidx_hbm.at[0, wid], src_v, isem[0]).wait()
    pltpu.make_async_copy(eidx_hbm.at[1, wid], dst_v, isem[1]).wait()
    plsc.subcore_barrier()

    _edge_loop(y_sh, src_v, dst_v, rows, acc_sh, gsem, ssem)

    plsc.subcore_barrier()
    pltpu.sync_copy(acc_sh.at[pl.ds(row0, RPT)], stage)
    pltpu.sync_copy(stage, out_hbm.at[c, pl.ds(row0, RPT)])

    @pl.when(c == 0)
    def _():
        pltpu.make_async_copy(yn_v, ynext_hbm.at[pl.ds(row0, RPT)],
                              isem[2]).wait()


def _prep_body(cnt_ref, x_ref, w1_ref, y1_ref, dis_ref):
    cnt = cnt_ref[0] + cnt_ref[1]
    deg = cnt + 1.0
    dis = lax.rsqrt(deg)
    xw = jnp.dot(x_ref[...], w1_ref[...], preferred_element_type=jnp.float32)
    y1 = xw * dis[:N, None]
    y1_ref[...] = jnp.concatenate(
        [y1, jnp.zeros((NP - N, H), jnp.float32)], axis=0)
    dis_ref[...] = dis


def _head_body(acc_ref, y_ref, cnt_ref, b3_ref, batch_ref,
               wm1_ref, bm1_ref, wm2_ref, bm2_ref, o_ref):
    cnt = cnt_ref[0] + cnt_ref[1]
    dis = lax.rsqrt(cnt[:N] + 1.0)[:, None]
    h3 = (acc_ref[0, :N] + acc_ref[1, :N] + y_ref[:N]) * dis + b3_ref[...]
    gid = lax.broadcasted_iota(jnp.int32, (G, N), 0)
    onehot = (gid == batch_ref[...][None, :]).astype(jnp.float32)
    sums = jnp.dot(onehot, h3, preferred_element_type=jnp.float32)
    counts = jnp.sum(onehot, axis=1)
    pooled = sums / jnp.maximum(counts, 1.0)[:, None]
    z = jnp.maximum(
        jnp.dot(pooled, wm1_ref[...], preferred_element_type=jnp.float32)
        + bm1_ref[...], 0.0)
    z = jnp.dot(z, wm2_ref[...],
                preferred_element_type=jnp.float32) + bm2_ref[...]
    m = jnp.max(z, axis=-1, keepdims=True)
    e = z - m
    o_ref[...] = e - jnp.log(jnp.sum(jnp.exp(e), axis=-1, keepdims=True))


_prep = pl.pallas_call(
    _prep_body,
    out_shape=(jax.ShapeDtypeStruct((NP, H), jnp.float32),
               jax.ShapeDtypeStruct((NP,), jnp.float32)),
)

_head = pl.pallas_call(
    _head_body,
    out_shape=jax.ShapeDtypeStruct((G, 10), jnp.float32),
)


def kernel(x, edge_index, batch, W1, b1, W2, b2, W3, b3, Wm1, bm1, Wm2, bm2):
    ei = edge_index.astype(jnp.int32)
    pad = jnp.full((2, NW * EPW - E), PAD_NODE, jnp.int32)
    eidx = jnp.concatenate([ei, pad], axis=1).reshape(2, NW, NCHUNK, CH)
    cnt = _deg_kernel(eidx)
    y1, dis = _prep(cnt, x, W1)
    acc1 = _edge_kernel(y1, eidx)
    acc2, y2 = _emid_kernel(y1, acc1, dis, b1, W2, eidx)
    acc3, y3 = _emid_kernel(y2, acc2, dis, b2, W3, eidx)
    return _head(acc3, y3, cnt, b3, batch.astype(jnp.int32), Wm1, bm1, Wm2, bm2)


# speedup vs baseline: 1.1907x; 1.1907x over previous
import functools

import jax
import jax.numpy as jnp
from jax import lax
from jax.experimental import pallas as pl
from jax.experimental.pallas import tpu as pltpu
from jax.experimental.pallas import tpu_sc as plsc

N = 10000
NP = 10240
E = 320000
G = 64
H = 16
NC = 2
NS = 16
NW = NC * NS
EPW = E // NW
CH = 80
NCHUNK = EPW // CH
RPT = NP // NS

_mesh = plsc.VectorSubcoreMesh(core_axis_name="c", subcore_axis_name="s")
_sc_params = pltpu.CompilerParams(use_tc_tiling_on_sc=False)


@functools.partial(
    pl.kernel,
    out_type=jax.ShapeDtypeStruct((NC, NP), jnp.float32),
    mesh=_mesh,
    compiler_params=_sc_params,
    scratch_types=[
        pltpu.VMEM((NCHUNK, CH), jnp.int32),
        pltpu.VMEM((CH,), jnp.float32),
        pltpu.VMEM((RPT,), jnp.float32),
        pltpu.VMEM_SHARED((NP,), jnp.float32),
    ] + [pltpu.SemaphoreType.DMA] * 8,
)
def _deg_kernel(eidx_hbm, out_hbm, dst_v, ones_v, stage_v, acc_sh, *ssem):
    c = lax.axis_index("c")
    s = lax.axis_index("s")

    def fill(i, _):
        ones_v[pl.ds(i * 16, 16)] = jnp.ones((16,), jnp.float32)
        return 0

    lax.fori_loop(0, CH // 16, fill, 0)

    def zero(i, _):
        stage_v[pl.ds(i * 16, 16)] = jnp.zeros((16,), jnp.float32)
        return 0

    lax.fori_loop(0, RPT // 16, zero, 0)
    pltpu.sync_copy(stage_v, acc_sh.at[pl.ds(s * RPT, RPT)])
    plsc.subcore_barrier()

    wid = s * NC + c
    pltpu.sync_copy(eidx_hbm.at[1, wid], dst_v)

    def body(g, _):
        for b in range(8):
            j = g * 8 + b

            @pl.when(j < NCHUNK)
            def _():
                @pl.when(j >= 8)
                def _():
                    pltpu.make_async_copy(
                        ones_v, acc_sh.at[dst_v.at[0]], ssem[b]).wait()

                pltpu.async_copy(
                    ones_v, acc_sh.at[dst_v.at[j]], ssem[b], add=True)

        return 0

    lax.fori_loop(0, (NCHUNK + 7) // 8, body, 0)
    for b in range(8):
        pltpu.make_async_copy(ones_v, acc_sh.at[dst_v.at[0]], ssem[b]).wait()
    plsc.subcore_barrier()
    pltpu.sync_copy(acc_sh.at[pl.ds(s * RPT, RPT)], stage_v)
    pltpu.sync_copy(stage_v, out_hbm.at[c, pl.ds(s * RPT, RPT)])


def _edge_loop(y_ref, src_v, dst_v, rows, acc_sh, gsem, ssem):
    def wait_g(b):
        pltpu.make_async_copy(y_ref.at[src_v.at[0]], rows.at[b],
                              gsem[b]).wait()

    def wait_s(b):
        pltpu.make_async_copy(rows.at[b], acc_sh.at[dst_v.at[0]],
                              ssem[b]).wait()

    for b in range(6):
        pltpu.async_copy(y_ref.at[src_v.at[b]], rows.at[b], gsem[b])

    def body(g, _):
        for b in range(12):
            j = g * 12 + b
            bf = (b + 6) % 12

            @pl.when(j + 6 < NCHUNK)
            def _():
                @pl.when(j >= 6)
                def _():
                    wait_s(bf)

                pltpu.async_copy(
                    y_ref.at[src_v.at[j + 6]], rows.at[bf], gsem[bf])

            @pl.when(j < NCHUNK)
            def _():
                wait_g(b)
                pltpu.async_copy(
                    rows.at[b], acc_sh.at[dst_v.at[j]], ssem[b], add=True)

        return 0

    lax.fori_loop(0, (NCHUNK + 11) // 12, body, 0)
    for b in range(12):
        wait_s(b)


@functools.partial(
    pl.kernel,
    out_type=jax.ShapeDtypeStruct((NC, NP, H), jnp.float32),
    mesh=_mesh,
    compiler_params=_sc_params,
    scratch_types=[
        pltpu.VMEM((NCHUNK, CH), jnp.int32),
        pltpu.VMEM((NCHUNK, CH), jnp.int32),
        pltpu.VMEM((12, CH, H), jnp.float32),
        pltpu.VMEM((RPT, H), jnp.float32),
        pltpu.VMEM_SHARED((NP, H), jnp.float32),
    ] + [pltpu.SemaphoreType.DMA] * 26,
)
def _edge_kernel(y_hbm, eidx_hbm, out_hbm,
                 src_v, dst_v, rows, stage, acc_sh, *sems):
    gsem = sems[:12]
    ssem = sems[12:24]
    isem = sems[24:]
    c = lax.axis_index("c")
    s = lax.axis_index("s")

    wid = s * NC + c
    pltpu.async_copy(eidx_hbm.at[0, wid], src_v, isem[0])
    pltpu.async_copy(eidx_hbm.at[1, wid], dst_v, isem[1])

    def zero(i, _):
        stage[i, :] = jnp.zeros((H,), jnp.float32)
        return 0

    lax.fori_loop(0, RPT, zero, 0)
    pltpu.sync_copy(stage, acc_sh.at[pl.ds(s * RPT, RPT)])
    pltpu.make_async_copy(eidx_hbm.at[0, wid], src_v, isem[0]).wait()
    pltpu.make_async_copy(eidx_hbm.at[1, wid], dst_v, isem[1]).wait()
    plsc.subcore_barrier()

    _edge_loop(y_hbm, src_v, dst_v, rows, acc_sh, gsem, ssem)

    plsc.subcore_barrier()
    pltpu.sync_copy(acc_sh.at[pl.ds(s * RPT, RPT)], stage)
    pltpu.sync_copy(stage, out_hbm.at[c, pl.ds(s * RPT, RPT)])


@functools.partial(
    pl.kernel,
    out_type=(jax.ShapeDtypeStruct((NC, NP, H), jnp.float32),
              jax.ShapeDtypeStruct((NP, H), jnp.float32)),
    mesh=_mesh,
    compiler_params=_sc_params,
    scratch_types=[
        pltpu.VMEM((NCHUNK, CH), jnp.int32),
        pltpu.VMEM((NCHUNK, CH), jnp.int32),
        pltpu.VMEM((12, CH, H), jnp.float32),
        pltpu.VMEM((RPT, H), jnp.float32),
        pltpu.VMEM((RPT, H), jnp.float32),
        pltpu.VMEM((RPT, H), jnp.float32),
        pltpu.VMEM((RPT, H), jnp.float32),
        pltpu.VMEM((RPT, H), jnp.float32),
        pltpu.VMEM((RPT,), jnp.float32),
        pltpu.VMEM((16,), jnp.float32),
        pltpu.VMEM((H, H), jnp.float32),
        pltpu.VMEM_SHARED((NP, H), jnp.float32),
        pltpu.VMEM_SHARED((NP, H), jnp.float32),
    ] + [pltpu.SemaphoreType.DMA] * 27,
)
def _emid_kernel(yp_hbm, acc_hbm, dis_hbm, b_hbm, w_hbm, eidx_hbm,
                 out_hbm, ynext_hbm,
                 src_v, dst_v, rows, stage, yp_v, a0_v, a1_v, yn_v,
                 dis_v, b_v, w_v, y_sh, acc_sh, *sems):
    gsem = sems[:12]
    ssem = sems[12:24]
    isem = sems[24:]
    c = lax.axis_index("c")
    s = lax.axis_index("s")
    wid = s * NC + c
    row0 = s * RPT

    pltpu.async_copy(eidx_hbm.at[0, wid], src_v, isem[0])
    pltpu.async_copy(eidx_hbm.at[1, wid], dst_v, isem[1])
    pltpu.sync_copy(yp_hbm.at[pl.ds(row0, RPT)], yp_v)
    pltpu.sync_copy(acc_hbm.at[0, pl.ds(row0, RPT)], a0_v)
    pltpu.sync_copy(acc_hbm.at[1, pl.ds(row0, RPT)], a1_v)
    pltpu.sync_copy(dis_hbm.at[pl.ds(row0, RPT)], dis_v)
    pltpu.sync_copy(b_hbm, b_v)
    pltpu.sync_copy(w_hbm, w_v)
    wrows = [w_v[k, :] for k in range(H)]
    bvec = b_v[...]

    def dense(gi, _):
        i0 = gi * 16
        dvec = dis_v[pl.ds(i0, 16)]
        for r in range(16):
            i = i0 + r
            d = dvec[r]
            a = a0_v[i, :] + a1_v[i, :] + yp_v[i, :]
            h = jnp.maximum(a * d + bvec, 0.0)
            o = h[0] * wrows[0]
            for k in range(1, H):
                o = o + h[k] * wrows[k]
            yn_v[i, :] = o * d
        return 0

    lax.fori_loop(0, RPT // 16, dense, 0)
    pltpu.sync_copy(yn_v, y_sh.at[pl.ds(row0, RPT)])

    @pl.when(c == 0)
    def _():
        pltpu.async_copy(yn_v, ynext_hbm.at[pl.ds(row0, RPT)], isem[2])

    def zero(i, _):
        stage[i, :] = jnp.zeros((H,), jnp.float32)
        return 0

    lax.fori_loop(0, RPT, zero, 0)
    pltpu.sync_copy(stage, acc_sh.at[pl.ds(row0, RPT)])
    pltpu.make_async_copy(eidx_hbm.at[0, wid], src_v, isem[0]).wait()
    pltpu.make_async_copy(eidx_hbm.at[1, wid], dst_v, isem[1]).wait()
    plsc.subcore_barrier()

    _edge_loop(y_sh, src_v, dst_v, rows, acc_sh, gsem, ssem)

    plsc.subcore_barrier()
    pltpu.sync_copy(acc_sh.at[pl.ds(row0, RPT)], stage)
    pltpu.sync_copy(stage, out_hbm.at[c, pl.ds(row0, RPT)])

    @pl.when(c == 0)
    def _():
        pltpu.make_async_copy(yn_v, ynext_hbm.at[pl.ds(row0, RPT)],
                              isem[2]).wait()


def _prep_body(cnt_ref, x_ref, w1_ref, y1_ref, dis_ref):
    cnt = cnt_ref[0] + cnt_ref[1]
    deg = cnt + 1.0
    dis = lax.rsqrt(deg)
    xw = jnp.dot(x_ref[...], w1_ref[...], preferred_element_type=jnp.float32)
    y1 = xw * dis[:N, None]
    y1_ref[...] = jnp.concatenate(
        [y1, jnp.zeros((NP - N, H), jnp.float32)], axis=0)
    dis_ref[...] = dis


def _head_body(acc_ref, y_ref, cnt_ref, b3_ref, batch_ref,
               wm1_ref, bm1_ref, wm2_ref, bm2_ref, o_ref):
    cnt = cnt_ref[0] + cnt_ref[1]
    dis = lax.rsqrt(cnt[:N] + 1.0)[:, None]
    h3 = (acc_ref[0, :N] + acc_ref[1, :N] + y_ref[:N]) * dis + b3_ref[...]
    gid = lax.broadcasted_iota(jnp.int32, (G, N), 0)
    onehot = (gid == batch_ref[...][None, :]).astype(jnp.float32)
    sums = jnp.dot(onehot, h3, preferred_element_type=jnp.float32)
    counts = jnp.sum(onehot, axis=1)
    pooled = sums / jnp.maximum(counts, 1.0)[:, None]
    z = jnp.maximum(
        jnp.dot(pooled, wm1_ref[...], preferred_element_type=jnp.float32)
        + bm1_ref[...], 0.0)
    z = jnp.dot(z, wm2_ref[...],
                preferred_element_type=jnp.float32) + bm2_ref[...]
    m = jnp.max(z, axis=-1, keepdims=True)
    e = z - m
    o_ref[...] = e - jnp.log(jnp.sum(jnp.exp(e), axis=-1, keepdims=True))


_prep = pl.pallas_call(
    _prep_body,
    out_shape=(jax.ShapeDtypeStruct((NP, H), jnp.float32),
               jax.ShapeDtypeStruct((NP,), jnp.float32)),
)

_head = pl.pallas_call(
    _head_body,
    out_shape=jax.ShapeDtypeStruct((G, 10), jnp.float32),
)


def kernel(x, edge_index, batch, W1, b1, W2, b2, W3, b3, Wm1, bm1, Wm2, bm2):
    eidx = edge_index.astype(jnp.int32).reshape(2, NW, NCHUNK, CH)
    cnt = _deg_kernel(eidx)
    y1, dis = _prep(cnt, x, W1)
    acc1 = _edge_kernel(y1, eidx)
    acc2, y2 = _emid_kernel(y1, acc1, dis, b1, W2, eidx)
    acc3, y3 = _emid_kernel(y2, acc2, dis, b2, W3, eidx)
    return _head(acc3, y3, cnt, b3, batch.astype(jnp.int32), Wm1, bm1, Wm2, bm2)
